# baseline (speedup 1.0000x reference)
import numpy as np
import jax
import jax.numpy as jnp
from jax import lax
from jax.experimental import pallas as pl
from jax.experimental.pallas import tpu as pltpu
from jax.experimental.pallas import tpu_sc as plsc

N = 100000
C = 128
NP = 100352
BN = 2048
NW = 32
WF = NP // NW
S = 64
S4 = 4 * S
NCH = WF // S

_PERM = np.empty((C,), np.int32)
for _g in range(4):
    for _j in range(32):
        _PERM[32 * _g + _j] = 32 * _g + (_j // 2 if _j % 2 == 0
                                         else 16 + _j // 2)


def _mm_body(x_ref, w_ref, b_ref, t_ref):
    xb = x_ref[...]
    dn = (((1,), (1,)), ((), ()))
    y0 = lax.dot_general(xb, w_ref[0], dn,
                         preferred_element_type=jnp.float32)
    y1 = lax.dot_general(xb, w_ref[1], dn,
                         preferred_element_type=jnp.float32)
    t_ref[0] = y0 + b_ref[...]
    t_ref[1] = y1


def _project(xv, wr, b2):
    return pl.pallas_call(
        _mm_body,
        grid=(NP // BN,),
        in_specs=[pl.BlockSpec((BN, C), lambda i: (i, 0)),
                  pl.BlockSpec((2, C, C), lambda i: (0, 0, 0)),
                  pl.BlockSpec((1, C), lambda i: (0, 0))],
        out_specs=pl.BlockSpec((2, BN, C), lambda i: (0, i, 0)),
        out_shape=jax.ShapeDtypeStruct((2, NP, C), jnp.float32),
    )(xv, wr, b2)


def _gather_body(t_hbm, g_hbm, pm_hbm, out_hbm,
                 gv0, gv1, jv00, jv01, jv10, jv11,
                 rows0, rows1, outv0, outv1, pmv,
                 semg0, semg1):
    gvs = (gv0, gv1)
    jvs = ((jv00, jv01), (jv10, jv11))
    rowss = (rows0, rows1)
    outvs = (outv0, outv1)
    sems = (semg0, semg1)
    cid = lax.axis_index("c")
    sid = lax.axis_index("s")
    wid = sid * 2 + cid

    pltpu.sync_copy(pm_hbm, pmv)
    pm16 = pmv[...]

    def f_of(ci):
        return jnp.minimum(wid * WF + ci * S, N - S)

    def fire(ci, b):
        f0 = f_of(ci)
        pltpu.sync_copy(g_hbm.at[pl.ds(4 * f0, S4)], gvs[b])
        for p in range(2):
            for u in range(8):
                sl = pl.ds(16 * (8 * p + u), 16)
                jvs[b][p][pl.ds(16 * u, 16)] = gvs[b][sl] + pm16
        for p in range(2):
            pltpu.async_copy(t_hbm.at[jvs[b][p]], rowss[b].at[p],
                             sems[b])

    def drain(b):
        for p in range(2):
            pltpu.make_async_copy(
                t_hbm.at[jvs[b][p]], rowss[b].at[p], sems[b]).wait()

    def flush(ci, b):
        rows = rowss[b]
        outv = outvs[b]

        for p in range(2):
            for sm in range(32):
                for g in range(8):
                    sl = pl.ds(16 * g, 16)
                    outv[32 * p + sm, sl] = (
                        rows[p, 4 * sm, sl] + rows[p, 4 * sm + 1, sl]
                        + rows[p, 4 * sm + 2, sl] + rows[p, 4 * sm + 3, sl])
        pltpu.sync_copy(outv, out_hbm.at[pl.ds(f_of(ci), S)])

    fire(0, 0)

    def step(i, _):
        c0 = 2 * i
        fire(c0 + 1, 1)
        drain(0)
        flush(c0, 0)
        fire(c0 + 2, 0)
        drain(1)
        flush(c0 + 1, 1)
        return 0

    lax.fori_loop(0, (NCH + 1) // 2, step, 0)
    drain(0)


_SC_CACHE = {}


def _sc_gather(tf, gflat, pm):
    if "k" not in _SC_CACHE:
        _SC_CACHE["k"] = pl.kernel(
            _gather_body,
            out_type=jax.ShapeDtypeStruct((N, C), jnp.float32),
            mesh=plsc.VectorSubcoreMesh(core_axis_name="c",
                                        subcore_axis_name="s"),
            scratch_types=[
                pltpu.VMEM((S4,), jnp.int32),
                pltpu.VMEM((S4,), jnp.int32),
                pltpu.VMEM((C,), jnp.int32),
                pltpu.VMEM((C,), jnp.int32),
                pltpu.VMEM((C,), jnp.int32),
                pltpu.VMEM((C,), jnp.int32),
                pltpu.VMEM((2, 128, C), jnp.float32),
                pltpu.VMEM((2, 128, C), jnp.float32),
                pltpu.VMEM((S, C), jnp.float32),
                pltpu.VMEM((S, C), jnp.float32),
                pltpu.VMEM((16,), jnp.int32),
                pltpu.SemaphoreType.DMA,
                pltpu.SemaphoreType.DMA,
            ],
        )
    return _SC_CACHE["k"](tf, gflat, pm)


def kernel(x, G, W, b):
    xv = jnp.transpose(x.reshape(C, N))
    wr = jnp.transpose(W[:, :, 0, :], (2, 0, 1))
    b2 = b.reshape(1, C)
    pm = jnp.tile(jnp.array([0, NP, NP, NP], jnp.int32), 4)
    t = _project(xv, wr, b2)
    tf = t.reshape(2 * NP, C)
    y = _sc_gather(tf, G.reshape(4 * N), pm)
    return jnp.transpose(y).reshape(1, C, N, 1)


# speedup vs baseline: 1.3977x; 1.3977x over previous
import numpy as np
import jax
import jax.numpy as jnp
from jax import lax
from jax.experimental import pallas as pl
from jax.experimental.pallas import tpu as pltpu
from jax.experimental.pallas import tpu_sc as plsc

N = 100000
C = 128
NP = 100352
BN = 2048
NW = 32
WF = NP // NW
S = 64
S4 = 4 * S
NCH = WF // S

_PERM = np.empty((C,), np.int32)
for _g in range(4):
    for _j in range(32):
        _PERM[32 * _g + _j] = 32 * _g + (_j // 2 if _j % 2 == 0
                                         else 16 + _j // 2)


def _mm_body(x_ref, w_ref, b_ref, t_ref):
    xb = x_ref[...]
    dn = (((1,), (1,)), ((), ()))
    y0 = lax.dot_general(xb, w_ref[0], dn,
                         preferred_element_type=jnp.float32)
    y1 = lax.dot_general(xb, w_ref[1], dn,
                         preferred_element_type=jnp.float32)
    t_ref[0] = y0 + b_ref[...]
    t_ref[1] = y1


def _project(xv, wr, b2):
    return pl.pallas_call(
        _mm_body,
        grid=(NP // BN,),
        in_specs=[pl.BlockSpec((BN, C), lambda i: (i, 0)),
                  pl.BlockSpec((2, C, C), lambda i: (0, 0, 0)),
                  pl.BlockSpec((1, C), lambda i: (0, 0))],
        out_specs=pl.BlockSpec((2, BN, C), lambda i: (0, i, 0)),
        out_shape=jax.ShapeDtypeStruct((2, NP, C), jnp.float32),
    )(xv, wr, b2)


def _gather_body(t_hbm, g_hbm, pm_hbm, out_hbm,
                 gv0, gv1, jv00, jv01, jv10, jv11,
                 rows0, rows1, outv0, outv1, pmv,
                 semg0, semg1):
    gvs = (gv0, gv1)
    jvs = ((jv00, jv01), (jv10, jv11))
    rowss = (rows0, rows1)
    outvs = (outv0, outv1)
    sems = (semg0, semg1)
    cid = lax.axis_index("c")
    sid = lax.axis_index("s")
    wid = sid * 2 + cid

    pltpu.sync_copy(pm_hbm, pmv)
    pm16 = pmv[...]

    def f_of(ci):
        return jnp.minimum(wid * WF + ci * S, N - S)

    def fire(ci, b):
        f0 = f_of(ci)
        pltpu.sync_copy(g_hbm.at[pl.ds(4 * f0, S4)], gvs[b])
        for p in range(2):
            for u in range(8):
                sl = pl.ds(16 * (8 * p + u), 16)
                jvs[b][p][pl.ds(16 * u, 16)] = gvs[b][sl] + pm16
        for p in range(2):
            pltpu.async_copy(t_hbm.at[jvs[b][p]], rowss[b].at[p],
                             sems[b])

    def drain(b):
        for p in range(2):
            pltpu.make_async_copy(
                t_hbm.at[jvs[b][p]], rowss[b].at[p], sems[b]).wait()

    def flush(ci, b):
        rows = rowss[b]
        outv = outvs[b]

        for p in range(2):
            def face(sm, _, p=p):
                for g in range(8):
                    sl = pl.ds(16 * g, 16)
                    outv[32 * p + sm, sl] = (
                        rows[p, 4 * sm, sl] + rows[p, 4 * sm + 1, sl]
                        + rows[p, 4 * sm + 2, sl] + rows[p, 4 * sm + 3, sl])
                return 0

            lax.fori_loop(0, 32, face, 0)
        pltpu.sync_copy(outv, out_hbm.at[pl.ds(f_of(ci), S)])

    fire(0, 0)

    def step(i, _):
        c0 = 2 * i
        fire(c0 + 1, 1)
        drain(0)
        flush(c0, 0)
        fire(c0 + 2, 0)
        drain(1)
        flush(c0 + 1, 1)
        return 0

    lax.fori_loop(0, (NCH + 1) // 2, step, 0)
    drain(0)


_SC_CACHE = {}


def _sc_gather(tf, gflat, pm):
    if "k" not in _SC_CACHE:
        _SC_CACHE["k"] = pl.kernel(
            _gather_body,
            out_type=jax.ShapeDtypeStruct((N, C), jnp.float32),
            mesh=plsc.VectorSubcoreMesh(core_axis_name="c",
                                        subcore_axis_name="s"),
            scratch_types=[
                pltpu.VMEM((S4,), jnp.int32),
                pltpu.VMEM((S4,), jnp.int32),
                pltpu.VMEM((C,), jnp.int32),
                pltpu.VMEM((C,), jnp.int32),
                pltpu.VMEM((C,), jnp.int32),
                pltpu.VMEM((C,), jnp.int32),
                pltpu.VMEM((2, 128, C), jnp.float32),
                pltpu.VMEM((2, 128, C), jnp.float32),
                pltpu.VMEM((S, C), jnp.float32),
                pltpu.VMEM((S, C), jnp.float32),
                pltpu.VMEM((16,), jnp.int32),
                pltpu.SemaphoreType.DMA,
                pltpu.SemaphoreType.DMA,
            ],
        )
    return _SC_CACHE["k"](tf, gflat, pm)


def kernel(x, G, W, b):
    xv = jnp.transpose(x.reshape(C, N))
    wr = jnp.transpose(W[:, :, 0, :], (2, 0, 1))
    b2 = b.reshape(1, C)
    pm = jnp.tile(jnp.array([0, NP, NP, NP], jnp.int32), 4)
    t = _project(xv, wr, b2)
    tf = t.reshape(2 * NP, C)
    y = _sc_gather(tf, G.reshape(4 * N), pm)
    return jnp.transpose(y).reshape(1, C, N, 1)


# speedup vs baseline: 1.4228x; 1.0180x over previous
import numpy as np
import jax
import jax.numpy as jnp
from jax import lax
from jax.experimental import pallas as pl
from jax.experimental.pallas import tpu as pltpu
from jax.experimental.pallas import tpu_sc as plsc

N = 100000
C = 128
NP = 100352
BN = 2048
NW = 32
WF = NP // NW
S = 64
S4 = 4 * S
NCH = WF // S

_PERM = np.empty((C,), np.int32)
for _g in range(4):
    for _j in range(32):
        _PERM[32 * _g + _j] = 32 * _g + (_j // 2 if _j % 2 == 0
                                         else 16 + _j // 2)


def _mm_body(x_ref, w_ref, b_ref, t_ref):
    xb = x_ref[...]
    dn = (((1,), (1,)), ((), ()))
    y0 = lax.dot_general(xb, w_ref[0], dn,
                         preferred_element_type=jnp.float32)
    y1 = lax.dot_general(xb, w_ref[1], dn,
                         preferred_element_type=jnp.float32)
    t_ref[0] = y0 + b_ref[...]
    t_ref[1] = y1


def _project(xv, wr, b2):
    return pl.pallas_call(
        _mm_body,
        grid=(NP // BN,),
        in_specs=[pl.BlockSpec((BN, C), lambda i: (i, 0)),
                  pl.BlockSpec((2, C, C), lambda i: (0, 0, 0)),
                  pl.BlockSpec((1, C), lambda i: (0, 0))],
        out_specs=pl.BlockSpec((2, BN, C), lambda i: (0, i, 0)),
        out_shape=jax.ShapeDtypeStruct((2, NP, C), jnp.float32),
    )(xv, wr, b2)


def _gather_body(t_hbm, g_hbm, pm_hbm, out_hbm, dum_hbm,
                 gv0, gv1, gv2, jv00, jv01, jv10, jv11, jv20, jv21,
                 rows0, rows1, rows2, outv0, outv1, outv2, pmv,
                 semg0, semg1, semg2, semo0, semo1, semo2):
    gvs = (gv0, gv1, gv2)
    jvs = ((jv00, jv01), (jv10, jv11), (jv20, jv21))
    rowss = (rows0, rows1, rows2)
    outvs = (outv0, outv1, outv2)
    sems = (semg0, semg1, semg2)
    semos = (semo0, semo1, semo2)
    cid = lax.axis_index("c")
    sid = lax.axis_index("s")
    wid = sid * 2 + cid

    pltpu.sync_copy(pm_hbm, pmv)
    pm16 = pmv[...]

    def f_of(ci):
        return jnp.minimum(wid * WF + ci * S, N - S)

    def fire(ci, b):
        f0 = f_of(ci)
        pltpu.sync_copy(g_hbm.at[pl.ds(4 * f0, S4)], gvs[b])
        for p in range(2):
            for u in range(8):
                sl = pl.ds(16 * (8 * p + u), 16)
                jvs[b][p][pl.ds(16 * u, 16)] = gvs[b][sl] + pm16
        for p in range(2):
            pltpu.async_copy(t_hbm.at[jvs[b][p]], rowss[b].at[p],
                             sems[b])

    def drain(b):
        for p in range(2):
            pltpu.make_async_copy(
                t_hbm.at[jvs[b][p]], rowss[b].at[p], sems[b]).wait()

    def flush(ci, b):
        rows = rowss[b]
        outv = outvs[b]
        pltpu.make_async_copy(outv, dum_hbm, semos[b]).wait()

        for p in range(2):
            def face(sm, _, p=p):
                for g in range(8):
                    sl = pl.ds(16 * g, 16)
                    outv[32 * p + sm, sl] = (
                        rows[p, 4 * sm, sl] + rows[p, 4 * sm + 1, sl]
                        + rows[p, 4 * sm + 2, sl] + rows[p, 4 * sm + 3, sl])
                return 0

            lax.fori_loop(0, 32, face, 0)
        pltpu.async_copy(outv, out_hbm.at[pl.ds(f_of(ci), S)], semos[b])

    for b in range(3):
        pltpu.async_copy(outvs[b], dum_hbm, semos[b])
    fire(0, 0)
    fire(1, 1)

    def step(i, _):
        c0 = 3 * i
        fire(c0 + 2, 2)
        drain(0)
        flush(c0, 0)
        fire(c0 + 3, 0)
        drain(1)
        flush(c0 + 1, 1)
        fire(c0 + 4, 1)
        drain(2)
        flush(c0 + 2, 2)
        return 0

    lax.fori_loop(0, (NCH + 2) // 3, step, 0)
    drain(0)
    drain(1)
    for b in range(3):
        pltpu.make_async_copy(outvs[b], dum_hbm, semos[b]).wait()


_SC_CACHE = {}


def _sc_gather(tf, gflat, pm):
    if "k" not in _SC_CACHE:
        _SC_CACHE["k"] = pl.kernel(
            _gather_body,
            out_type=[jax.ShapeDtypeStruct((N, C), jnp.float32),
                      jax.ShapeDtypeStruct((S, C), jnp.float32)],
            mesh=plsc.VectorSubcoreMesh(core_axis_name="c",
                                        subcore_axis_name="s"),
            scratch_types=[
                pltpu.VMEM((S4,), jnp.int32),
                pltpu.VMEM((S4,), jnp.int32),
                pltpu.VMEM((S4,), jnp.int32),
                pltpu.VMEM((C,), jnp.int32),
                pltpu.VMEM((C,), jnp.int32),
                pltpu.VMEM((C,), jnp.int32),
                pltpu.VMEM((C,), jnp.int32),
                pltpu.VMEM((C,), jnp.int32),
                pltpu.VMEM((C,), jnp.int32),
                pltpu.VMEM((2, 128, C), jnp.float32),
                pltpu.VMEM((2, 128, C), jnp.float32),
                pltpu.VMEM((2, 128, C), jnp.float32),
                pltpu.VMEM((S, C), jnp.float32),
                pltpu.VMEM((S, C), jnp.float32),
                pltpu.VMEM((S, C), jnp.float32),
                pltpu.VMEM((16,), jnp.int32),
                pltpu.SemaphoreType.DMA,
                pltpu.SemaphoreType.DMA,
                pltpu.SemaphoreType.DMA,
                pltpu.SemaphoreType.DMA,
                pltpu.SemaphoreType.DMA,
                pltpu.SemaphoreType.DMA,
            ],
        )
    return _SC_CACHE["k"](tf, gflat, pm)[0]


def kernel(x, G, W, b):
    xv = jnp.transpose(x.reshape(C, N))
    wr = jnp.transpose(W[:, :, 0, :], (2, 0, 1))
    b2 = b.reshape(1, C)
    pm = jnp.tile(jnp.array([0, NP, NP, NP], jnp.int32), 4)
    t = _project(xv, wr, b2)
    tf = t.reshape(2 * NP, C)
    y = _sc_gather(tf, G.reshape(4 * N), pm)
    return jnp.transpose(y).reshape(1, C, N, 1)


# speedup vs baseline: 1.5610x; 1.0971x over previous
import numpy as np
import jax
import jax.numpy as jnp
from jax import lax
from jax.experimental import pallas as pl
from jax.experimental.pallas import tpu as pltpu
from jax.experimental.pallas import tpu_sc as plsc

N = 100000
C = 128
NP = 100352
BN = 2048
NW = 32
WF = NP // NW
S = 64
S4 = 4 * S
NCH = WF // S

_PERM = np.empty((C,), np.int32)
for _g in range(4):
    for _j in range(32):
        _PERM[32 * _g + _j] = 32 * _g + (_j // 2 if _j % 2 == 0
                                         else 16 + _j // 2)


def _mm_body(x_ref, w_ref, b_ref, t_ref):
    xb = x_ref[...]
    dn = (((1,), (1,)), ((), ()))
    y0 = lax.dot_general(xb, w_ref[0], dn,
                         preferred_element_type=jnp.float32)
    y1 = lax.dot_general(xb, w_ref[1], dn,
                         preferred_element_type=jnp.float32)
    t_ref[0] = y0 + b_ref[...]
    t_ref[1] = y1


def _project(xv, wr, b2):
    return pl.pallas_call(
        _mm_body,
        grid=(NP // BN,),
        in_specs=[pl.BlockSpec((BN, C), lambda i: (i, 0)),
                  pl.BlockSpec((2, C, C), lambda i: (0, 0, 0)),
                  pl.BlockSpec((1, C), lambda i: (0, 0))],
        out_specs=pl.BlockSpec((2, BN, C), lambda i: (0, i, 0)),
        out_shape=jax.ShapeDtypeStruct((2, NP, C), jnp.float32),
    )(xv, wr, b2)


def _gather_body(t_hbm, g_hbm, pm_hbm, out_hbm, dum_hbm,
                 gall, jv00, jv01, jv10, jv11,
                 rows0, rows1, outv0, outv1, pmv,
                 semg0, semg1, semo0, semo1):
    jvs = ((jv00, jv01), (jv10, jv11))
    rowss = (rows0, rows1)
    outvs = (outv0, outv1)
    sems = (semg0, semg1)
    semos = (semo0, semo1)
    cid = lax.axis_index("c")
    sid = lax.axis_index("s")
    wid = sid * 2 + cid

    pltpu.sync_copy(pm_hbm, pmv)
    pm16 = pmv[...]
    gbase = jnp.minimum(wid * (4 * WF), 4 * N - 4 * WF)
    pltpu.sync_copy(g_hbm.at[pl.ds(gbase, 4 * WF)], gall)

    def f_of(ci):
        return jnp.minimum(wid * WF + ci * S, N - S)

    def fire(ci, b):
        loff = 4 * f_of(ci) - gbase
        for p in range(2):
            for u in range(8):
                sl = pl.ds(loff + 16 * (8 * p + u), 16)
                jvs[b][p][pl.ds(16 * u, 16)] = gall[sl] + pm16
        for p in range(2):
            pltpu.async_copy(t_hbm.at[jvs[b][p]], rowss[b].at[p],
                             sems[b])

    def drain(b):
        for p in range(2):
            pltpu.make_async_copy(
                t_hbm.at[jvs[b][p]], rowss[b].at[p], sems[b]).wait()

    def flush(ci, b):
        rows = rowss[b]
        outv = outvs[b]
        pltpu.make_async_copy(outv, dum_hbm, semos[b]).wait()

        for p in range(2):
            def face(sm, _, p=p):
                for g in range(8):
                    sl = pl.ds(16 * g, 16)
                    outv[32 * p + sm, sl] = (
                        rows[p, 4 * sm, sl] + rows[p, 4 * sm + 1, sl]
                        + rows[p, 4 * sm + 2, sl] + rows[p, 4 * sm + 3, sl])
                return 0

            lax.fori_loop(0, 32, face, 0)
        pltpu.async_copy(outv, out_hbm.at[pl.ds(f_of(ci), S)], semos[b])

    for b in range(2):
        pltpu.async_copy(outvs[b], dum_hbm, semos[b])
    fire(0, 0)

    def step(i, _):
        c0 = 2 * i
        fire(c0 + 1, 1)
        drain(0)
        flush(c0, 0)
        fire(c0 + 2, 0)
        drain(1)
        flush(c0 + 1, 1)
        return 0

    lax.fori_loop(0, (NCH + 1) // 2, step, 0)
    drain(0)
    for b in range(2):
        pltpu.make_async_copy(outvs[b], dum_hbm, semos[b]).wait()


_SC_CACHE = {}


def _sc_gather(tf, gflat, pm):
    if "k" not in _SC_CACHE:
        _SC_CACHE["k"] = pl.kernel(
            _gather_body,
            out_type=[jax.ShapeDtypeStruct((N, C), jnp.float32),
                      jax.ShapeDtypeStruct((S, C), jnp.float32)],
            mesh=plsc.VectorSubcoreMesh(core_axis_name="c",
                                        subcore_axis_name="s"),
            scratch_types=[
                pltpu.VMEM((4 * WF,), jnp.int32),
                pltpu.VMEM((C,), jnp.int32),
                pltpu.VMEM((C,), jnp.int32),
                pltpu.VMEM((C,), jnp.int32),
                pltpu.VMEM((C,), jnp.int32),
                pltpu.VMEM((2, 128, C), jnp.float32),
                pltpu.VMEM((2, 128, C), jnp.float32),
                pltpu.VMEM((S, C), jnp.float32),
                pltpu.VMEM((S, C), jnp.float32),
                pltpu.VMEM((16,), jnp.int32),
                pltpu.SemaphoreType.DMA,
                pltpu.SemaphoreType.DMA,
                pltpu.SemaphoreType.DMA,
                pltpu.SemaphoreType.DMA,
            ],
        )
    return _SC_CACHE["k"](tf, gflat, pm)[0]


def kernel(x, G, W, b):
    xv = jnp.transpose(x.reshape(C, N))
    wr = jnp.transpose(W[:, :, 0, :], (2, 0, 1))
    b2 = b.reshape(1, C)
    pm = jnp.tile(jnp.array([0, NP, NP, NP], jnp.int32), 4)
    t = _project(xv, wr, b2)
    tf = t.reshape(2 * NP, C)
    y = _sc_gather(tf, G.reshape(4 * N), pm)
    return jnp.transpose(y).reshape(1, C, N, 1)
